# baseline (speedup 1.0000x reference)
import functools

import jax
import jax.numpy as jnp
from jax import lax
from jax.experimental import pallas as pl
from jax.experimental.pallas import tpu as pltpu

_F32 = jnp.float32


def _dotg(a, b, ca, cb):
    return lax.dot_general(a, b, (((ca,), (cb,)), ((), ())),
                           preferred_element_type=_F32)


def _gcn_fold_kernel(a_ref, x_ref, w0_ref, b0t_ref, w1_ref, w0g_ref,
                     z_ref, c_ref):
    t = pl.program_id(1)

    @pl.when(t == 0)
    def _():
        c_ref[...] = jnp.dot(w1_ref[0], w0g_ref[0],
                             preferred_element_type=_F32)

    t1 = _dotg(x_ref[...], a_ref[0], 0, 1)
    h1t = jnp.maximum(_dotg(w0_ref[0], t1, 0, 0) + b0t_ref[0], 0.0)
    z_ref[0] = _dotg(h1t, c_ref[...], 0, 0)


def _head_kernel(a_ref, z_ref, x_ref, w0x_ref, bt_ref, mw0_ref, mb0_ref,
                 wl_ref, bl_ref, o_ref, *, num_graph_types):
    acc = jnp.dot(x_ref[...], w0x_ref[...], preferred_element_type=_F32)
    acc += bt_ref[...]
    for g in range(num_graph_types):
        acc += jnp.dot(a_ref[g], z_ref[g], preferred_element_type=_F32)
    hm = jnp.maximum(acc, 0.0)
    hm = jnp.maximum(
        jnp.dot(hm, mw0_ref[...], preferred_element_type=_F32)
        + mb0_ref[...], 0.0)
    o_ref[...] = jnp.sum(hm * wl_ref[...], axis=1, keepdims=True) + bl_ref[...]


def kernel(a_hats, x, gnn_w_0, gnn_w_1, gnn_b_0, gnn_b_1, w0x, w0g, b0,
           mlp_w_0, mlp_b_0, mlp_w_1, mlp_b_1):
    G, N, _ = a_hats.shape
    Fin = x.shape[1]
    H = gnn_w_0.shape[2]
    H0 = w0x.shape[1]
    Hm = mlp_w_0.shape[1]

    bias_tot = b0
    for g in range(G):
        bias_tot = bias_tot + jnp.dot(gnn_b_1[g], w0g[g],
                                      preferred_element_type=_F32)
    b0t = jnp.transpose(gnn_b_0, (0, 2, 1))

    r1 = 256 if N % 256 == 0 else N
    t1 = N // r1
    z = pl.pallas_call(
        _gcn_fold_kernel,
        out_shape=jax.ShapeDtypeStruct((G, N, H0), _F32),
        grid=(G, t1),
        in_specs=[
            pl.BlockSpec((1, r1, N), lambda g, t: (g, t, 0)),
            pl.BlockSpec((N, Fin), lambda g, t: (0, 0)),
            pl.BlockSpec((1, Fin, H), lambda g, t: (g, 0, 0)),
            pl.BlockSpec((1, H, 1), lambda g, t: (g, 0, 0)),
            pl.BlockSpec((1,) + gnn_w_1.shape[1:], lambda g, t: (g, 0, 0)),
            pl.BlockSpec((1,) + w0g.shape[1:], lambda g, t: (g, 0, 0)),
        ],
        out_specs=pl.BlockSpec((1, r1, H0), lambda g, t: (g, t, 0)),
        scratch_shapes=[pltpu.VMEM((H, H0), _F32)],
        compiler_params=pltpu.CompilerParams(
            dimension_semantics=("arbitrary", "arbitrary"),
            vmem_limit_bytes=64 * 2**20),
        cost_estimate=pl.CostEstimate(
            flops=int(G * (2 * N * N * Fin + 2 * N * Fin * H
                           + 2 * N * H * H0)),
            transcendentals=0,
            bytes_accessed=int(4 * (G * N * N + G * N * H0 + N * Fin))),
    )(a_hats, x, gnn_w_0, b0t, gnn_w_1, w0g)

    r2 = 256 if N % 256 == 0 else N
    t2 = N // r2
    out = pl.pallas_call(
        functools.partial(_head_kernel, num_graph_types=G),
        out_shape=jax.ShapeDtypeStruct((N, 1), _F32),
        grid=(t2,),
        in_specs=[
            pl.BlockSpec((G, r2, N), lambda t: (0, t, 0)),
            pl.BlockSpec((G, N, H0), lambda t: (0, 0, 0)),
            pl.BlockSpec((r2, Fin), lambda t: (t, 0)),
            pl.BlockSpec((Fin, H0), lambda t: (0, 0)),
            pl.BlockSpec((1, H0), lambda t: (0, 0)),
            pl.BlockSpec((H0, Hm), lambda t: (0, 0)),
            pl.BlockSpec((1, Hm), lambda t: (0, 0)),
            pl.BlockSpec((1, Hm), lambda t: (0, 0)),
            pl.BlockSpec((1, 1), lambda t: (0, 0)),
        ],
        out_specs=pl.BlockSpec((r2, 1), lambda t: (t, 0)),
        compiler_params=pltpu.CompilerParams(
            dimension_semantics=("arbitrary",),
            vmem_limit_bytes=64 * 2**20),
        cost_estimate=pl.CostEstimate(
            flops=int(G * 2 * N * N * H0 + 2 * N * Fin * H0
                      + 2 * N * H0 * Hm),
            transcendentals=0,
            bytes_accessed=int(4 * (G * N * N + G * N * H0 + N * Fin + N))),
    )(a_hats, z, x, w0x, bias_tot, mlp_w_0, mlp_b_0, mlp_w_1, mlp_b_1)
    return out


# speedup vs baseline: 1.5320x; 1.5320x over previous
import functools

import jax
import jax.numpy as jnp
from jax import lax
from jax.experimental import pallas as pl
from jax.experimental.pallas import tpu as pltpu

_F32 = jnp.float32


def _dotg(a, b, ca, cb):
    return lax.dot_general(a, b, (((ca,), (cb,)), ((), ())),
                           preferred_element_type=_F32)


def _tabgnn_kernel(a_ref, x_ref, w0_ref, b0t_ref, w1_ref, w0g_ref, w0x_ref,
                   bt_ref, mw0_ref, mb0_ref, wl_ref, bl_ref, o_ref, acc_ref,
                   *, num_graph_types):
    g = pl.program_id(0)
    a = a_ref[0]

    t1 = _dotg(x_ref[...], a, 0, 1)
    h1t = jnp.maximum(_dotg(w0_ref[0], t1, 0, 0) + b0t_ref[0], 0.0)
    hw1t = _dotg(w1_ref[0], h1t, 0, 0)
    embt = _dotg(hw1t, a, 1, 1)

    @pl.when(g == 0)
    def _():
        acc_ref[...] = (jnp.dot(x_ref[...], w0x_ref[...],
                                preferred_element_type=_F32) + bt_ref[...])

    acc_ref[...] += _dotg(embt, w0g_ref[0], 0, 0)

    @pl.when(g == num_graph_types - 1)
    def _():
        hm = jnp.maximum(acc_ref[...], 0.0)
        hm = jnp.maximum(
            jnp.dot(hm, mw0_ref[...], preferred_element_type=_F32)
            + mb0_ref[...], 0.0)
        o_ref[...] = (jnp.sum(hm * wl_ref[...], axis=1, keepdims=True)
                      + bl_ref[...])


def kernel(a_hats, x, gnn_w_0, gnn_w_1, gnn_b_0, gnn_b_1, w0x, w0g, b0,
           mlp_w_0, mlp_b_0, mlp_w_1, mlp_b_1):
    G, N, _ = a_hats.shape
    Fin = x.shape[1]
    H = gnn_w_0.shape[2]
    H0 = w0x.shape[1]
    Hm = mlp_w_0.shape[1]

    bias_tot = b0
    for g in range(G):
        bias_tot = bias_tot + jnp.dot(gnn_b_1[g], w0g[g],
                                      preferred_element_type=_F32)
    b0t = jnp.transpose(gnn_b_0, (0, 2, 1))

    flops = int(G * (2 * N * N * Fin + 2 * N * Fin * H + 2 * N * H * Hm
                     + 2 * N * N * Hm + 2 * N * Hm * H0)
                + 2 * N * Fin * H0 + 2 * N * H0 * Hm + 2 * N * Hm)
    bytes_accessed = int(4 * (G * N * N + N * Fin + N
                              + Fin * H0 + G * (Fin * H + H + H * Hm + Hm * H0)))

    out = pl.pallas_call(
        functools.partial(_tabgnn_kernel, num_graph_types=G),
        out_shape=jax.ShapeDtypeStruct((N, 1), _F32),
        grid=(G,),
        in_specs=[
            pl.BlockSpec((1, N, N), lambda g: (g, 0, 0)),
            pl.BlockSpec((N, Fin), lambda g: (0, 0)),
            pl.BlockSpec((1, Fin, H), lambda g: (g, 0, 0)),
            pl.BlockSpec((1, H, 1), lambda g: (g, 0, 0)),
            pl.BlockSpec((1,) + gnn_w_1.shape[1:], lambda g: (g, 0, 0)),
            pl.BlockSpec((1,) + w0g.shape[1:], lambda g: (g, 0, 0)),
            pl.BlockSpec((Fin, H0), lambda g: (0, 0)),
            pl.BlockSpec((1, H0), lambda g: (0, 0)),
            pl.BlockSpec((H0, Hm), lambda g: (0, 0)),
            pl.BlockSpec((1, Hm), lambda g: (0, 0)),
            pl.BlockSpec((1, Hm), lambda g: (0, 0)),
            pl.BlockSpec((1, 1), lambda g: (0, 0)),
        ],
        out_specs=pl.BlockSpec((N, 1), lambda g: (0, 0)),
        scratch_shapes=[pltpu.VMEM((N, H0), _F32)],
        compiler_params=pltpu.CompilerParams(
            dimension_semantics=("arbitrary",),
            vmem_limit_bytes=58 * 2**20),
        cost_estimate=pl.CostEstimate(flops=flops, transcendentals=0,
                                      bytes_accessed=bytes_accessed),
    )(a_hats, x, gnn_w_0, b0t, gnn_w_1, w0g, w0x, bias_tot,
      mlp_w_0, mlp_b_0, mlp_w_1, mlp_b_1)
    return out
